# baseline (speedup 1.0000x reference)
import functools

import jax
import jax.numpy as jnp
from jax import lax
from jax.experimental import pallas as pl
from jax.experimental.pallas import tpu as pltpu
from jax.experimental.pallas import tpu_sc as plsc

_LANES = 16
_NCHUNK = 4


def _sc_mesh():
    return plsc.VectorSubcoreMesh(core_axis_name="c", subcore_axis_name="s")


def _num_workers():
    info = plsc.get_sparse_core_info()
    return info.num_cores * info.num_subcores, info.num_cores


_SC_PARAMS = pltpu.CompilerParams(needs_layout_passes=False)


def _ones_where(mask):
    return jnp.where(
        mask, jnp.ones((_LANES,), jnp.int32), jnp.zeros((_LANES,), jnp.int32)
    )


def _mask_nonzero(masks_flat, n_active):
    S, V = masks_flat.shape
    n_chunks = V // _LANES
    _, NC = _num_workers()

    @functools.partial(
        pl.kernel,
        out_type=jax.ShapeDtypeStruct((S, n_active), jnp.int32),
        mesh=_sc_mesh(),
        scratch_types=[
            pltpu.VMEM((1, V), jnp.float32),
            pltpu.VMEM((1, n_active), jnp.int32),
        ],
        compiler_params=_SC_PARAMS,
    )
    def body(masks_hbm, mi_hbm, mrow, midx):
        wid = lax.axis_index("s") * NC + lax.axis_index("c")

        @pl.when(wid < S)
        def _():
            pltpu.sync_copy(masks_hbm.at[pl.ds(wid, 1)], mrow)
            zeros16 = jnp.zeros((_LANES,), jnp.int32)
            lanes = lax.iota(jnp.int32, _LANES)

            @plsc.parallel_loop(
                0, n_chunks, step=1, unroll=4,
                carry=jnp.zeros((_LANES,), jnp.int32),
            )
            def step(c, base):
                mv = mrow[0, pl.ds(c * _LANES, _LANES)]
                act = mv > 0.5
                pc = plsc.cumsum(_ones_where(act))
                pos = base + pc - 1
                vox = c * _LANES + lanes
                plsc.store_scatter(midx, [zeros16, pos], vox, mask=act)
                return base + plsc.all_reduce_population_count(act)

            pltpu.sync_copy(midx, mi_hbm.at[pl.ds(wid, 1)])

    return body(masks_flat)


def _gather_chunk(flat_c, mask_idx, ids_c):
    Bc, V = flat_c.shape
    S, K = mask_idx.shape
    NW, NC = _num_workers()
    rows_per = Bc // NW
    k_steps = K // _LANES
    assert Bc % NW == 0 and Bc % _LANES == 0

    @functools.partial(
        pl.kernel,
        out_type=jax.ShapeDtypeStruct((Bc, K), jnp.float32),
        mesh=_sc_mesh(),
        scratch_types=[
            pltpu.VMEM((1, V), jnp.float32),
            pltpu.VMEM((1, V), jnp.float32),
            pltpu.VMEM((1, K), jnp.int32),
            pltpu.VMEM((1, K), jnp.int32),
            pltpu.VMEM((1, K), jnp.float32),
            pltpu.VMEM((1, K), jnp.float32),
            pltpu.VMEM((Bc,), jnp.int32),
            pltpu.SemaphoreType.DMA,
            pltpu.SemaphoreType.DMA,
            pltpu.SemaphoreType.DMA,
            pltpu.SemaphoreType.DMA,
            pltpu.SemaphoreType.DMA,
            pltpu.SemaphoreType.DMA,
        ],
        compiler_params=_SC_PARAMS,
    )
    def body(flat_hbm, mi_hbm, ids_hbm, x_hbm, row0, row1, midx0, midx1,
             xr0, xr1, sid_v, sr0, sr1, si0, si1, sw0, sw1):
        sub = lax.axis_index("s")
        core = lax.axis_index("c")
        wid = sub * NC + core
        base = wid * rows_per
        pltpu.sync_copy(ids_hbm, sid_v)
        rows = (row0, row1)
        midxs = (midx0, midx1)
        xrs = (xr0, xr1)
        srs = (sr0, sr1)
        sis = (si0, si1)
        sws = (sw0, sw1)
        zeros16 = jnp.zeros((_LANES,), jnp.int32)
        lanes = lax.iota(jnp.int32, _LANES)

        def start(j):
            p = j % 2
            r = base + j
            vvec = sid_v[pl.ds((r // _LANES) * _LANES, _LANES)]
            sid = jnp.sum(jnp.where(lanes == r % _LANES, vvec, zeros16))
            rd = pltpu.async_copy(flat_hbm.at[pl.ds(r, 1)], rows[p], srs[p])
            ird = pltpu.async_copy(mi_hbm.at[pl.ds(sid, 1)], midxs[p], sis[p])
            return rd, ird

        pending = {0: start(0)}
        writebacks = {}
        for j in range(rows_per):
            if j + 1 < rows_per:
                pending[j + 1] = start(j + 1)
            rd, ird = pending.pop(j)
            rd.wait()
            ird.wait()
            if j - 2 in writebacks:
                writebacks.pop(j - 2).wait()
            p = j % 2

            @plsc.parallel_loop(0, k_steps, step=1, unroll=8)
            def _(k, _p=p):
                iv = midxs[_p][0, pl.ds(k * _LANES, _LANES)]
                xrs[_p][0, pl.ds(k * _LANES, _LANES)] = plsc.load_gather(
                    rows[_p], [zeros16, iv]
                )

            writebacks[j] = pltpu.async_copy(
                xrs[p], x_hbm.at[pl.ds(base + j, 1)], sws[p]
            )
        for j in sorted(writebacks):
            writebacks.pop(j).wait()

    return body(flat_c, mask_idx, ids_c)


def _expert_matmul(Xs, Ws, bs, ids):
    nch = len(Xs)
    Bc, K = Xs[0].shape
    B = Bc * nch
    S, OUT, _ = Ws.shape
    ids3 = ids.reshape(1, B, 1)

    def body(*refs):
        x_refs = refs[:nch]
        w_ref, b_ref, id_ref, o_ref = refs[nch:]
        s = pl.program_id(0)
        w = w_ref[0]
        bias = b_ref[0]
        for c in range(nch):
            acc = lax.dot_general(
                x_refs[c][...],
                w,
                (((1,), (1,)), ((), ())),
                precision=lax.Precision.DEFAULT,
                preferred_element_type=jnp.float32,
            )
            res = acc + bias
            keep = id_ref[0, c * Bc:(c + 1) * Bc] == s
            sl = slice(c * Bc, (c + 1) * Bc)
            o_ref[sl, :] = jnp.where(keep, res, o_ref[sl, :])

    out = pl.pallas_call(
        body,
        grid=(S,),
        in_specs=[pl.BlockSpec((Bc, K), lambda s: (0, 0)) for _ in range(nch)]
        + [
            pl.BlockSpec((1, OUT, K), lambda s: (s, 0, 0)),
            pl.BlockSpec((1, 1, OUT), lambda s: (s, 0, 0)),
            pl.BlockSpec((1, B, 1), lambda s: (0, 0, 0)),
        ],
        out_specs=pl.BlockSpec((B, OUT), lambda s: (0, 0)),
        out_shape=jax.ShapeDtypeStruct((B, OUT), jnp.float32),
    )(*Xs, Ws, bs.astype(jnp.float32).reshape(S, 1, OUT), ids3)
    return out


def kernel(id_batch, fmri_batch, masks, Ws, bs):
    B = fmri_batch.shape[0]
    S, OUT, K = Ws.shape
    masks_flat = masks.reshape(S, -1)
    ids = id_batch.astype(jnp.int32)
    Bc = B // _NCHUNK

    mask_idx = _mask_nonzero(masks_flat, K)
    Xs = []
    for c in range(_NCHUNK):
        flat_c = fmri_batch[c * Bc:(c + 1) * Bc].reshape(Bc, -1)
        Xs.append(_gather_chunk(flat_c, mask_idx, ids[c * Bc:(c + 1) * Bc]))
    out = _expert_matmul(Xs, Ws, bs, ids)
    return out.reshape(B, 1, OUT)


# speedup vs baseline: 1.4933x; 1.4933x over previous
import functools

import jax
import jax.numpy as jnp
from jax import lax
from jax.experimental import pallas as pl
from jax.experimental.pallas import tpu as pltpu
from jax.experimental.pallas import tpu_sc as plsc

_LANES = 16


def _sc_mesh():
    return plsc.VectorSubcoreMesh(core_axis_name="c", subcore_axis_name="s")


def _num_workers():
    info = plsc.get_sparse_core_info()
    return info.num_cores * info.num_subcores, info.num_cores


_SC_PARAMS = pltpu.CompilerParams(needs_layout_passes=False)


def _ones_where(mask):
    return jnp.where(
        mask, jnp.ones((_LANES,), jnp.int32), jnp.zeros((_LANES,), jnp.int32)
    )


def _mask_and_gather(masks_flat, flat, ids, K):
    S, V = masks_flat.shape
    B, _ = flat.shape
    NW, NC = _num_workers()
    rows_per = B // NW
    k_steps = K // _LANES
    n_chunks = V // _LANES

    assert rows_per == _LANES

    @functools.partial(
        pl.kernel,
        out_type=jax.ShapeDtypeStruct((B, K), jnp.float32),
        mesh=_sc_mesh(),
        scratch_types=[
            pltpu.VMEM((1, V), jnp.float32),
            pltpu.VMEM((1, V), jnp.float32),
            pltpu.VMEM((1, K), jnp.int32),
            pltpu.VMEM((1, K), jnp.int32),
            pltpu.VMEM((1, K), jnp.float32),
            pltpu.VMEM((1, K), jnp.float32),
            pltpu.VMEM((rows_per,), jnp.int32),
            pltpu.VMEM_SHARED((S, K), jnp.int32),
            pltpu.SemaphoreType.DMA,
            pltpu.SemaphoreType.DMA,
            pltpu.SemaphoreType.DMA,
            pltpu.SemaphoreType.DMA,
            pltpu.SemaphoreType.DMA,
            pltpu.SemaphoreType.DMA,
        ],
        compiler_params=_SC_PARAMS,
    )
    def body(masks_hbm, flat_hbm, ids_hbm, x_hbm, row0, row1, midx0, midx1,
             xr0, xr1, sid_v, shared_mi, sr0, sr1, si0, si1, sw0, sw1):
        sub = lax.axis_index("s")
        core = lax.axis_index("c")
        wid = sub * NC + core
        base = wid * rows_per
        pltpu.sync_copy(ids_hbm.at[pl.ds(base, rows_per)], sid_v)

        @pl.when(sub < S)
        def _():
            pltpu.sync_copy(masks_hbm.at[pl.ds(sub, 1)], row0)
            zeros16 = jnp.zeros((_LANES,), jnp.int32)
            lanes = lax.iota(jnp.int32, _LANES)

            @plsc.parallel_loop(
                0, n_chunks, step=1, unroll=4,
                carry=jnp.zeros((_LANES,), jnp.int32),
            )
            def step(c, bpos):
                mv = row0[0, pl.ds(c * _LANES, _LANES)]
                act = mv > 0.5
                pc = plsc.cumsum(_ones_where(act))
                pos = bpos + pc - 1
                vox = c * _LANES + lanes
                plsc.store_scatter(midx0, [zeros16, pos], vox, mask=act)
                return bpos + plsc.all_reduce_population_count(act)

            pltpu.sync_copy(midx0, shared_mi.at[pl.ds(sub, 1)])

        plsc.subcore_barrier()
        rows = (row0, row1)
        midxs = (midx0, midx1)
        xrs = (xr0, xr1)
        srs = (sr0, sr1)
        sis = (si0, si1)
        sws = (sw0, sw1)
        zeros16 = jnp.zeros((_LANES,), jnp.int32)
        lanes = lax.iota(jnp.int32, _LANES)
        vvec = sid_v[pl.ds(0, _LANES)]

        def start(j):
            p = j % 2
            sid = jnp.sum(jnp.where(lanes == j, vvec, zeros16))
            rd = pltpu.async_copy(flat_hbm.at[pl.ds(base + j, 1)], rows[p], srs[p])
            ird = pltpu.async_copy(shared_mi.at[pl.ds(sid, 1)], midxs[p], sis[p])
            return rd, ird

        pending = {0: start(0)}
        writebacks = {}
        for j in range(rows_per):
            if j + 1 < rows_per:
                pending[j + 1] = start(j + 1)
            rd, ird = pending.pop(j)
            rd.wait()
            ird.wait()
            if j - 2 in writebacks:
                writebacks.pop(j - 2).wait()
            p = j % 2

            @plsc.parallel_loop(0, k_steps, step=1, unroll=8)
            def _(k, _p=p):
                iv = midxs[_p][0, pl.ds(k * _LANES, _LANES)]
                xrs[_p][0, pl.ds(k * _LANES, _LANES)] = plsc.load_gather(
                    rows[_p], [zeros16, iv]
                )

            writebacks[j] = pltpu.async_copy(
                xrs[p], x_hbm.at[pl.ds(base + j, 1)], sws[p]
            )
        for j in sorted(writebacks):
            writebacks.pop(j).wait()

    return body(masks_flat, flat, ids)


def _expert_matmul(X, Ws, bs, ids):
    B, K = X.shape
    S, OUT, _ = Ws.shape
    H = S // 2
    ids3 = ids.reshape(1, B, 1)

    OH = OUT // 2

    def body(x_ref, w1_ref, w2_ref, b_ref, id_ref, o_ref):
        s = pl.program_id(1)
        idv = id_ref[0]
        x = x_ref[...]

        def commit(w_ref, e):
            acc = lax.dot_general(
                x,
                w_ref[0],
                (((1,), (1,)), ((), ())),
                precision=lax.Precision.DEFAULT,
                preferred_element_type=jnp.float32,
            )
            res = acc + b_ref[pl.ds(e, 1), 0]
            keep = idv == e
            o_ref[...] = jnp.where(keep, res, o_ref[...])

        commit(w1_ref, s)
        commit(w2_ref, s + H)

    out = pl.pallas_call(
        body,
        grid=(2, H),
        in_specs=[
            pl.BlockSpec((B, K), lambda h, s: (0, 0)),
            pl.BlockSpec((1, OH, K), lambda h, s: (s, h, 0)),
            pl.BlockSpec((1, OH, K), lambda h, s: (s + H, h, 0)),
            pl.BlockSpec((S, 1, OH), lambda h, s: (0, 0, h)),
            pl.BlockSpec((1, B, 1), lambda h, s: (0, 0, 0)),
        ],
        out_specs=pl.BlockSpec((B, OH), lambda h, s: (0, h)),
        out_shape=jax.ShapeDtypeStruct((B, OUT), jnp.float32),
    )(X, Ws, Ws, bs.astype(jnp.float32).reshape(S, 1, OUT), ids3)
    return out


def kernel(id_batch, fmri_batch, masks, Ws, bs):
    B = fmri_batch.shape[0]
    S, OUT, K = Ws.shape
    flat = fmri_batch.reshape(B, -1)
    masks_flat = masks.reshape(S, -1)
    ids = id_batch.astype(jnp.int32)

    X = _mask_and_gather(masks_flat, flat, ids, K)
    out = _expert_matmul(X, Ws, bs, ids)
    return out.reshape(B, 1, OUT)
